# baseline (speedup 1.0000x reference)
import functools

import jax
import jax.numpy as jnp
from jax import lax
from jax.experimental import pallas as pl
from jax.experimental.pallas import tpu as pltpu
from jax.experimental.pallas import tpu_sc as plsc

N_NODES = 10000
N_EDGES = 320000
D_FEAT = 128

NC = 2
NS = 16
NW = NC * NS
W = 40
EPW = 10240
WPW = EPW // W
NBUF = 8
CHUNK = 32
NCHUNK = WPW // CHUNK
N_EPAD = NW * EPW
N_ACC = 10008

ROWS_MAIN = 624
TAIL_BASE = NS * ROWS_MAIN
TAIL_OUT = N_NODES - TAIL_BASE
TAIL_ZERO = N_ACC - TAIL_BASE


def _sc_segment_partials(image, src1d, dst4d):
    mesh = plsc.VectorSubcoreMesh(core_axis_name="c", subcore_axis_name="s")

    @functools.partial(
        pl.kernel,
        out_type=jax.ShapeDtypeStruct((NC, N_NODES, D_FEAT), jnp.float32),
        mesh=mesh,
        compiler_params=pltpu.CompilerParams(use_tc_tiling_on_sc=False),
        scratch_types=[
            pltpu.VMEM_SHARED((N_ACC, D_FEAT), jnp.float32),
            pltpu.VMEM((CHUNK * W,), jnp.int32),
            pltpu.VMEM((CHUNK * W,), jnp.int32),
            pltpu.VMEM((CHUNK, W), jnp.int32),
            pltpu.VMEM((CHUNK, W), jnp.int32),
            pltpu.VMEM((NBUF, W, D_FEAT), jnp.float32),
            pltpu.SemaphoreType.DMA,
            pltpu.SemaphoreType.DMA,
            pltpu.SemaphoreType.DMA,
            pltpu.SemaphoreType.DMA,
            pltpu.SemaphoreType.DMA,
            pltpu.SemaphoreType.DMA,
            pltpu.SemaphoreType.DMA,
            pltpu.SemaphoreType.DMA,
            pltpu.SemaphoreType.DMA,
        ],
    )
    def k(image_hbm, src_hbm, dst_hbm, out_hbm, acc, src0, src1, dst0, dst1,
          ring, g0, g1, g2, g3, g4, g5, g6, g7, isem):
        c = lax.axis_index("c")
        s = lax.axis_index("s")
        wid = c * NS + s

        gsem = [g0, g1, g2, g3, g4, g5, g6, g7]
        srcb = [src0, src1]
        dstb = [dst0, dst1]

        zero = jnp.zeros((16,), jnp.float32)

        @pl.loop(0, W)
        def _(i):
            @pl.loop(0, D_FEAT, step=16)
            def _(j):
                ring[0, i, pl.ds(j, 16)] = zero

        row_base = s * ROWS_MAIN

        @pl.loop(0, ROWS_MAIN - 24, step=W)
        def _(r):
            pltpu.sync_copy(ring.at[0], acc.at[pl.ds(row_base + r, W)])

        pltpu.sync_copy(ring.at[0].at[pl.ds(0, 24)],
                        acc.at[pl.ds(row_base + 600, 24)])

        @pl.when(s == NS - 1)
        def _():
            pltpu.sync_copy(ring.at[0].at[pl.ds(0, TAIL_ZERO)],
                            acc.at[pl.ds(TAIL_BASE, TAIL_ZERO)])

        def load_chunk(ci, sb, db, sem):
            pltpu.async_copy(
                src_hbm.at[pl.ds(wid * EPW + ci * (CHUNK * W), CHUNK * W)],
                sb, sem)
            pltpu.async_copy(dst_hbm.at[wid, ci], db, sem)

        def wait_chunk(ci, sb, db, sem):
            pltpu.make_async_copy(
                src_hbm.at[pl.ds(wid * EPW + ci * (CHUNK * W), CHUNK * W)],
                sb, sem).wait()
            pltpu.make_async_copy(dst_hbm.at[wid, ci], db, sem).wait()

        load_chunk(0, src0, dst0, isem)
        wait_chunk(0, src0, dst0, isem)

        plsc.subcore_barrier()

        def start_gather(sb, widx, b):
            pltpu.async_copy(image_hbm.at[sb.at[pl.ds(widx * W, W)]],
                             ring.at[b], gsem[b])

        def wait_gather(b):
            pltpu.make_async_copy(image_hbm.at[src0.at[pl.ds(0, W)]],
                                  ring.at[b], gsem[b]).wait()

        def scatter_add(db, widx, b):
            pltpu.sync_copy(ring.at[b], acc.at[db.at[widx]], add=True)

        for b in range(NBUF):
            start_gather(src0, b, b)

        for ci in range(NCHUNK):
            sb, db = srcb[ci % 2], dstb[ci % 2]
            sn, dn = srcb[(ci + 1) % 2], dstb[(ci + 1) % 2]
            if ci + 1 < NCHUNK:
                load_chunk(ci + 1, sn, dn, isem)

            @pl.loop(0, CHUNK - NBUF, step=NBUF)
            def _(i):
                for b in range(NBUF):
                    wait_gather(b)
                    scatter_add(db, i + b, b)
                    start_gather(sb, i + b + NBUF, b)

            if ci + 1 < NCHUNK:
                wait_chunk(ci + 1, sn, dn, isem)
                for b in range(NBUF):
                    wait_gather(b)
                    scatter_add(db, CHUNK - NBUF + b, b)
                    start_gather(sn, b, b)
            else:
                for b in range(NBUF):
                    wait_gather(b)
                    scatter_add(db, CHUNK - NBUF + b, b)

        plsc.subcore_barrier()

        pltpu.sync_copy(
            acc.at[pl.ds(row_base, ROWS_MAIN)],
            out_hbm.at[c].at[pl.ds(row_base, ROWS_MAIN)],
        )

        @pl.when(s == NS - 1)
        def _():
            pltpu.sync_copy(
                acc.at[pl.ds(TAIL_BASE, TAIL_OUT)],
                out_hbm.at[c].at[pl.ds(TAIL_BASE, TAIL_OUT)],
            )

    return k(image, src1d, dst4d)


def _tc_combine(partials):
    def body(p_ref, o_ref):
        o_ref[...] = p_ref[0] + p_ref[1]

    blk = 2000
    return pl.pallas_call(
        body,
        out_shape=jax.ShapeDtypeStruct((N_NODES, D_FEAT), jnp.float32),
        grid=(N_NODES // blk,),
        in_specs=[pl.BlockSpec((NC, blk, D_FEAT), lambda i: (0, i, 0))],
        out_specs=pl.BlockSpec((blk, D_FEAT), lambda i: (i, 0)),
    )(partials)


@jax.jit
def kernel(image, edge_index):
    pad = N_EPAD - N_EDGES
    src1d = jnp.concatenate(
        [edge_index[0], jnp.zeros((pad,), jnp.int32)])
    dst4d = jnp.concatenate(
        [edge_index[1], jnp.full((pad,), N_NODES, jnp.int32)]
    ).reshape(NW, NCHUNK, CHUNK, W)
    partials = _sc_segment_partials(image, src1d, dst4d)
    mailbox_agg = _tc_combine(partials)
    return (image, mailbox_agg)


# speedup vs baseline: 3.8350x; 3.8350x over previous
import functools

import jax
import jax.numpy as jnp
from jax import lax
from jax.experimental import pallas as pl
from jax.experimental.pallas import tpu as pltpu
from jax.experimental.pallas import tpu_sc as plsc

N_NODES = 10000
N_EDGES = 320000
D_FEAT = 128

NC = 2
NS = 16
NW = NC * NS
W = 40
EPW = 10240
WPW = EPW // W
NBUF = 8
CHUNK = 32
NCHUNK = WPW // CHUNK
N_EPAD = NW * EPW
N_ACC = 10008

ROWS_MAIN = 624
TAIL_BASE = NS * ROWS_MAIN
TAIL_OUT = N_NODES - TAIL_BASE
TAIL_ZERO = N_ACC - TAIL_BASE


def _sc_segment_partials(image, src1d, dst4d):
    mesh = plsc.VectorSubcoreMesh(core_axis_name="c", subcore_axis_name="s")

    @functools.partial(
        pl.kernel,
        out_type=jax.ShapeDtypeStruct((NC, N_NODES, D_FEAT), jnp.float32),
        mesh=mesh,
        compiler_params=pltpu.CompilerParams(use_tc_tiling_on_sc=False),
        scratch_types=[
            pltpu.VMEM_SHARED((N_ACC, D_FEAT), jnp.float32),
            pltpu.VMEM((CHUNK * W,), jnp.int32),
            pltpu.VMEM((CHUNK * W,), jnp.int32),
            pltpu.VMEM((CHUNK, W), jnp.int32),
            pltpu.VMEM((CHUNK, W), jnp.int32),
            pltpu.VMEM((NBUF, W, D_FEAT), jnp.float32),
            pltpu.SemaphoreType.DMA,
            pltpu.SemaphoreType.DMA,
            pltpu.SemaphoreType.DMA,
            pltpu.SemaphoreType.DMA,
            pltpu.SemaphoreType.DMA,
            pltpu.SemaphoreType.DMA,
            pltpu.SemaphoreType.DMA,
            pltpu.SemaphoreType.DMA,
            pltpu.SemaphoreType.DMA,
        ],
    )
    def k(image_hbm, src_hbm, dst_hbm, out_hbm, acc, src0, src1, dst0, dst1,
          ring, g0, g1, g2, g3, g4, g5, g6, g7, isem):
        c = lax.axis_index("c")
        s = lax.axis_index("s")
        wid = c * NS + s

        gsem = [g0, g1, g2, g3, g4, g5, g6, g7]
        srcb = [src0, src1]
        dstb = [dst0, dst1]

        zero = jnp.zeros((16,), jnp.float32)

        @pl.loop(0, W)
        def _(i):
            @pl.loop(0, D_FEAT, step=16)
            def _(j):
                ring[0, i, pl.ds(j, 16)] = zero

        row_base = s * ROWS_MAIN

        @pl.loop(0, ROWS_MAIN - 24, step=W)
        def _(r):
            pltpu.sync_copy(ring.at[0], acc.at[pl.ds(row_base + r, W)])

        pltpu.sync_copy(ring.at[0].at[pl.ds(0, 24)],
                        acc.at[pl.ds(row_base + 600, 24)])

        @pl.when(s == NS - 1)
        def _():
            pltpu.sync_copy(ring.at[0].at[pl.ds(0, TAIL_ZERO)],
                            acc.at[pl.ds(TAIL_BASE, TAIL_ZERO)])

        def load_chunk(ci, sb, db, sem):
            pltpu.async_copy(
                src_hbm.at[pl.ds(wid * EPW + ci * (CHUNK * W), CHUNK * W)],
                sb, sem)
            pltpu.async_copy(dst_hbm.at[wid, ci], db, sem)

        def wait_chunk(ci, sb, db, sem):
            pltpu.make_async_copy(
                src_hbm.at[pl.ds(wid * EPW + ci * (CHUNK * W), CHUNK * W)],
                sb, sem).wait()
            pltpu.make_async_copy(dst_hbm.at[wid, ci], db, sem).wait()

        load_chunk(0, src0, dst0, isem)
        wait_chunk(0, src0, dst0, isem)

        plsc.subcore_barrier()

        def start_gather(sb, widx, b):
            pltpu.async_copy(image_hbm.at[sb.at[pl.ds(widx * W, W)]],
                             ring.at[b], gsem[b])

        def wait_gather(b):
            pltpu.make_async_copy(image_hbm.at[src0.at[pl.ds(0, W)]],
                                  ring.at[b], gsem[b]).wait()

        def scatter_add(db, widx, b):
            pltpu.sync_copy(ring.at[b], acc.at[db.at[widx]], add=True)

        for b in range(NBUF):
            start_gather(src0, b, b)

        for ci in range(NCHUNK):
            sb, db = srcb[ci % 2], dstb[ci % 2]
            sn, dn = srcb[(ci + 1) % 2], dstb[(ci + 1) % 2]
            if ci + 1 < NCHUNK:
                load_chunk(ci + 1, sn, dn, isem)

            @pl.loop(0, CHUNK - NBUF, step=NBUF)
            def _(i):
                for b in range(NBUF):
                    wait_gather(b)
                    scatter_add(db, i + b, b)
                    start_gather(sb, i + b + NBUF, b)

            if ci + 1 < NCHUNK:
                wait_chunk(ci + 1, sn, dn, isem)
                for b in range(NBUF):
                    wait_gather(b)
                    scatter_add(db, CHUNK - NBUF + b, b)
                    start_gather(sn, b, b)
            else:
                for b in range(NBUF):
                    wait_gather(b)
                    scatter_add(db, CHUNK - NBUF + b, b)

        plsc.subcore_barrier()

        pltpu.sync_copy(
            acc.at[pl.ds(row_base, ROWS_MAIN)],
            out_hbm.at[c].at[pl.ds(row_base, ROWS_MAIN)],
        )

        @pl.when(s == NS - 1)
        def _():
            pltpu.sync_copy(
                acc.at[pl.ds(TAIL_BASE, TAIL_OUT)],
                out_hbm.at[c].at[pl.ds(TAIL_BASE, TAIL_OUT)],
            )

    return k(image, src1d, dst4d)


def _tc_combine(partials):
    def body(p_ref, o_ref):
        o_ref[...] = p_ref[0] + p_ref[1]

    blk = 2000
    return pl.pallas_call(
        body,
        out_shape=jax.ShapeDtypeStruct((N_NODES, D_FEAT), jnp.float32),
        grid=(N_NODES // blk,),
        in_specs=[pl.BlockSpec((NC, blk, D_FEAT), lambda i: (0, i, 0))],
        out_specs=pl.BlockSpec((blk, D_FEAT), lambda i: (i, 0)),
    )(partials)


@jax.jit
def kernel(image, edge_index):
    ppw = EPW - N_EDGES // NW
    src_pad = jnp.broadcast_to((jnp.arange(ppw, dtype=jnp.int32) * 41)
                               % N_NODES, (NW, ppw))
    dst_pad = jnp.broadcast_to(N_NODES + (jnp.arange(ppw, dtype=jnp.int32)
                                          % (N_ACC - N_NODES)), (NW, ppw))
    src1d = jnp.concatenate(
        [edge_index[0].reshape(NW, N_EDGES // NW), src_pad], axis=1).reshape(-1)
    dst4d = jnp.concatenate(
        [edge_index[1].reshape(NW, N_EDGES // NW), dst_pad], axis=1
    ).reshape(NW, NCHUNK, CHUNK, W)
    partials = _sc_segment_partials(image, src1d, dst4d)
    mailbox_agg = _tc_combine(partials)
    return (image, mailbox_agg)


# speedup vs baseline: 4.0554x; 1.0575x over previous
import functools

import jax
import jax.numpy as jnp
from jax import lax
from jax.experimental import pallas as pl
from jax.experimental.pallas import tpu as pltpu
from jax.experimental.pallas import tpu_sc as plsc

N_NODES = 10000
N_EDGES = 320000
D_FEAT = 128

NC = 2
NS = 16
NW = NC * NS
W = 40
EPW = N_EDGES // NW
WPW = EPW // W
NBUF = 5

ROWS_MAIN = 624
TAIL_BASE = NS * ROWS_MAIN
TAIL_ROWS = N_NODES - TAIL_BASE


def _sc_segment_partials(image, src1d, dst3d):
    mesh = plsc.VectorSubcoreMesh(core_axis_name="c", subcore_axis_name="s")

    @functools.partial(
        pl.kernel,
        out_type=jax.ShapeDtypeStruct((NC, N_NODES, D_FEAT), jnp.float32),
        mesh=mesh,
        compiler_params=pltpu.CompilerParams(use_tc_tiling_on_sc=False),
        scratch_types=[
            pltpu.VMEM_SHARED((N_NODES, D_FEAT), jnp.float32),
            pltpu.VMEM((EPW,), jnp.int32),
            pltpu.VMEM((WPW, W), jnp.int32),
            pltpu.VMEM((NBUF, W, D_FEAT), jnp.float32),
            pltpu.SemaphoreType.DMA,
            pltpu.SemaphoreType.DMA,
            pltpu.SemaphoreType.DMA,
            pltpu.SemaphoreType.DMA,
            pltpu.SemaphoreType.DMA,
            pltpu.SemaphoreType.DMA,
        ],
    )
    def k(image_hbm, src_hbm, dst_hbm, out_hbm, acc, src_idx, dst_idx,
          ring, g0, g1, g2, g3, g4, isem):
        c = lax.axis_index("c")
        s = lax.axis_index("s")
        wid = c * NS + s

        gsem = [g0, g1, g2, g3, g4]

        zero = jnp.zeros((16,), jnp.float32)

        @pl.loop(0, W)
        def _(i):
            @pl.loop(0, D_FEAT, step=16)
            def _(j):
                ring[0, i, pl.ds(j, 16)] = zero

        row_base = s * ROWS_MAIN

        @pl.loop(0, ROWS_MAIN - 24, step=W)
        def _(r):
            pltpu.sync_copy(ring.at[0], acc.at[pl.ds(row_base + r, W)])

        pltpu.sync_copy(ring.at[0].at[pl.ds(0, 24)],
                        acc.at[pl.ds(row_base + 600, 24)])

        @pl.when(s == NS - 1)
        def _():
            pltpu.sync_copy(ring.at[0].at[pl.ds(0, TAIL_ROWS)],
                            acc.at[pl.ds(TAIL_BASE, TAIL_ROWS)])

        pltpu.async_copy(src_hbm.at[pl.ds(wid * EPW, EPW)], src_idx, isem)
        pltpu.async_copy(dst_hbm.at[wid], dst_idx, isem)
        pltpu.make_async_copy(src_hbm.at[pl.ds(wid * EPW, EPW)], src_idx,
                              isem).wait()
        pltpu.make_async_copy(dst_hbm.at[wid], dst_idx, isem).wait()

        plsc.subcore_barrier()

        def start_gather(widx, b):
            pltpu.async_copy(image_hbm.at[src_idx.at[pl.ds(widx * W, W)]],
                             ring.at[b], gsem[b])

        def wait_gather(b):
            pltpu.make_async_copy(image_hbm.at[src_idx.at[pl.ds(0, W)]],
                                  ring.at[b], gsem[b]).wait()

        def scatter_add(widx, b):
            pltpu.sync_copy(ring.at[b], acc.at[dst_idx.at[widx]], add=True)

        for b in range(NBUF):
            start_gather(b, b)

        @pl.loop(0, WPW - NBUF, step=NBUF)
        def _(i):
            for b in range(NBUF):
                wait_gather(b)
                scatter_add(i + b, b)
                start_gather(i + b + NBUF, b)

        for b in range(NBUF):
            wait_gather(b)
            scatter_add(WPW - NBUF + b, b)

        plsc.subcore_barrier()

        pltpu.sync_copy(
            acc.at[pl.ds(row_base, ROWS_MAIN)],
            out_hbm.at[c].at[pl.ds(row_base, ROWS_MAIN)],
        )

        @pl.when(s == NS - 1)
        def _():
            pltpu.sync_copy(
                acc.at[pl.ds(TAIL_BASE, TAIL_ROWS)],
                out_hbm.at[c].at[pl.ds(TAIL_BASE, TAIL_ROWS)],
            )

    return k(image, src1d, dst3d)


def _tc_combine(partials):
    def body(p_ref, o_ref):
        o_ref[...] = p_ref[0] + p_ref[1]

    blk = 2000
    return pl.pallas_call(
        body,
        out_shape=jax.ShapeDtypeStruct((N_NODES, D_FEAT), jnp.float32),
        grid=(N_NODES // blk,),
        in_specs=[pl.BlockSpec((NC, blk, D_FEAT), lambda i: (0, i, 0))],
        out_specs=pl.BlockSpec((blk, D_FEAT), lambda i: (i, 0)),
    )(partials)


@jax.jit
def kernel(image, edge_index):
    src1d = edge_index[0]
    dst3d = edge_index[1].reshape(NW, WPW, W)
    partials = _sc_segment_partials(image, src1d, dst3d)
    mailbox_agg = _tc_combine(partials)
    return (image, mailbox_agg)
